# Initial kernel scaffold; baseline (speedup 1.0000x reference)
import jax
import jax.numpy as jnp
from jax.experimental import pallas as pl


def kernel(user_ids, item_ids, user_emb_mf, item_emb_mf, user_emb_mlp, item_emb_mlp, W1, b1, W2, b2, W_out, b_out):
    raise NotImplementedError("write your pallas kernel here")


# speedup vs baseline: 1.3885x; 1.3885x over previous
import functools
import jax
import jax.numpy as jnp
from jax import lax
from jax.experimental import pallas as pl
from jax.experimental.pallas import tpu as pltpu
from jax.experimental.pallas import tpu_sc as plsc

_NC = 2
_NS = 16
_NW = _NC * _NS

_BATCH = 16384
_EMB = 64
_BPW = _BATCH // _NW


def _sc_gather_body(uid_hbm, iid_hbm, t_umf, t_imf, t_umlp, t_imlp,
                    o_umf, o_imf, o_umlp, o_imlp,
                    uidx_v, iidx_v, buf_a, buf_b, sem_a, sem_b):
    wid = lax.axis_index("s") * _NC + lax.axis_index("c")
    base = wid * _BPW
    pltpu.sync_copy(uid_hbm.at[pl.ds(base, _BPW)], uidx_v)
    pltpu.sync_copy(iid_hbm.at[pl.ds(base, _BPW)], iidx_v)

    cp_a = pltpu.async_copy(t_umf.at[uidx_v], buf_a, sem_a)
    cp_b = pltpu.async_copy(t_imf.at[iidx_v], buf_b, sem_b)
    cp_a.wait()
    pltpu.sync_copy(buf_a, o_umf.at[pl.ds(base, _BPW)])
    cp_a = pltpu.async_copy(t_umlp.at[uidx_v], buf_a, sem_a)
    cp_b.wait()
    pltpu.sync_copy(buf_b, o_imf.at[pl.ds(base, _BPW)])
    cp_b = pltpu.async_copy(t_imlp.at[iidx_v], buf_b, sem_b)
    cp_a.wait()
    pltpu.sync_copy(buf_a, o_umlp.at[pl.ds(base, _BPW)])
    cp_b.wait()
    pltpu.sync_copy(buf_b, o_imlp.at[pl.ds(base, _BPW)])


def _sc_gather(user_ids, item_ids, t_umf, t_imf, t_umlp, t_imlp):
    mesh = plsc.VectorSubcoreMesh(
        core_axis_name="c", subcore_axis_name="s",
        num_cores=_NC, num_subcores=_NS)
    row = jax.ShapeDtypeStruct((_BATCH, _EMB), jnp.float32)
    k = pl.kernel(
        _sc_gather_body,
        out_type=(row, row, row, row),
        mesh=mesh,
        scratch_types=[
            pltpu.VMEM((_BPW,), jnp.int32),
            pltpu.VMEM((_BPW,), jnp.int32),
            pltpu.VMEM((_BPW, _EMB), jnp.float32),
            pltpu.VMEM((_BPW, _EMB), jnp.float32),
            pltpu.SemaphoreType.DMA,
            pltpu.SemaphoreType.DMA,
        ],
    )
    return k(user_ids, item_ids, t_umf, t_imf, t_umlp, t_imlp)


_BLK = 1024


def _mlp_body(umf_ref, imf_ref, umlp_ref, imlp_ref,
              w1_ref, b1_ref, w2_ref, b2_ref, wo_ref, bo_ref, out_ref):
    h1 = jnp.dot(umlp_ref[...], w1_ref[0:_EMB, :],
                 preferred_element_type=jnp.float32)
    h1 += jnp.dot(imlp_ref[...], w1_ref[_EMB:2 * _EMB, :],
                  preferred_element_type=jnp.float32)
    h1 = jnp.maximum(h1 + b1_ref[...], 0.0)
    h2 = jnp.dot(h1, w2_ref[...], preferred_element_type=jnp.float32)
    h2 = jnp.maximum(h2 + b2_ref[...], 0.0)
    mf = umf_ref[...] * imf_ref[...]
    o = jnp.dot(mf, wo_ref[0:_EMB, :], preferred_element_type=jnp.float32)
    o += jnp.dot(h2, wo_ref[_EMB:, :], preferred_element_type=jnp.float32)
    out_ref[...] = o[:, 0] + bo_ref[0]


def _mlp(umf, imf, umlp, imlp, W1, b1, W2, b2, W_out, b_out):
    n_blk = _BATCH // _BLK
    row_spec = pl.BlockSpec((_BLK, _EMB), lambda i: (i, 0))
    full = lambda s: pl.BlockSpec(s, lambda i: tuple(0 for _ in s))
    return pl.pallas_call(
        _mlp_body,
        grid=(n_blk,),
        in_specs=[
            row_spec, row_spec, row_spec, row_spec,
            full(W1.shape), full(b1.shape), full(W2.shape), full(b2.shape),
            full(W_out.shape), full(b_out.shape),
        ],
        out_specs=pl.BlockSpec((_BLK,), lambda i: (i,)),
        out_shape=jax.ShapeDtypeStruct((_BATCH,), jnp.float32),
    )(umf, imf, umlp, imlp, W1, b1, W2, b2, W_out, b_out)


@jax.jit
def kernel(user_ids, item_ids, user_emb_mf, item_emb_mf, user_emb_mlp,
           item_emb_mlp, W1, b1, W2, b2, W_out, b_out):
    umf = jnp.take(user_emb_mf, user_ids, axis=0)
    imf = jnp.take(item_emb_mf, item_ids, axis=0)
    umlp = jnp.take(user_emb_mlp, user_ids, axis=0)
    imlp = jnp.take(item_emb_mlp, item_ids, axis=0)
    return _mlp(umf, imf, umlp, imlp, W1, b1, W2, b2, W_out, b_out)
